# Initial kernel scaffold; baseline (speedup 1.0000x reference)
import jax
import jax.numpy as jnp
from jax.experimental import pallas as pl


def kernel(x, adj, W, b):
    raise NotImplementedError("write your pallas kernel here")


# speedup vs baseline: 1.0369x; 1.0369x over previous
import jax
import jax.numpy as jnp
from jax.experimental import pallas as pl
from jax.experimental.pallas import tpu as pltpu

_BM = 400


def _gcn_body(adj_ref, x_ref, w_ref, b_ref, out_ref):
    ax = jnp.dot(adj_ref[...], x_ref[...], preferred_element_type=jnp.float32)
    out_ref[...] = (
        jnp.dot(ax, w_ref[...], preferred_element_type=jnp.float32) + b_ref[...]
    )


def kernel(x, adj, W, b):
    n, din = x.shape
    dout = W.shape[1]
    b2 = b.reshape(1, dout)
    return pl.pallas_call(
        _gcn_body,
        grid=(pl.cdiv(n, _BM),),
        in_specs=[
            pl.BlockSpec((_BM, n), lambda i: (i, 0)),
            pl.BlockSpec((n, din), lambda i: (0, 0)),
            pl.BlockSpec((din, dout), lambda i: (0, 0)),
            pl.BlockSpec((1, dout), lambda i: (0, 0)),
        ],
        out_specs=pl.BlockSpec((_BM, dout), lambda i: (i, 0)),
        out_shape=jax.ShapeDtypeStruct((n, dout), jnp.float32),
        compiler_params=pltpu.CompilerParams(
            dimension_semantics=("parallel",),
        ),
    )(adj, x, W, b2)
